# Initial kernel scaffold; baseline (speedup 1.0000x reference)
import jax
import jax.numpy as jnp
from jax.experimental import pallas as pl


def kernel(patches, centers):
    raise NotImplementedError("write your pallas kernel here")


# speedup vs baseline: 9.1431x; 9.1431x over previous
import functools

import numpy as np
import jax
import jax.numpy as jnp
from jax import lax
from jax.experimental import pallas as pl
from jax.experimental.pallas import tpu as pltpu
from jax.experimental.pallas import tpu_sc as plsc

_NC, _NS, _L = 2, 16, 16
_CH = 128


@functools.lru_cache(maxsize=None)
def _make_sc_kernel(bs, n, d_model, kpad):
    nch = kpad // _CH
    n_vecs = n // _L
    reps = bs // (_NC * _NS)
    mesh = plsc.VectorSubcoreMesh(
        core_axis_name="c", subcore_axis_name="s",
        num_cores=_NC, num_subcores=_NS)

    @functools.partial(
        pl.kernel,
        out_type=(
            jax.ShapeDtypeStruct((bs * kpad, d_model), jnp.float32),
            jax.ShapeDtypeStruct((bs, kpad * 3), jnp.float32),
        ),
        mesh=mesh,
        compiler_params=pltpu.CompilerParams(
            needs_layout_passes=False, use_tc_tiling_on_sc=False),
        scratch_types=[
            pltpu.VMEM((n * 3,), jnp.float32),
            pltpu.VMEM((bs,), jnp.int32),
            pltpu.VMEM((n,), jnp.int32),
            pltpu.VMEM((n,), jnp.int32),
            pltpu.VMEM((n,), jnp.int32),
            pltpu.VMEM((n,), jnp.int32),
            pltpu.VMEM((256,), jnp.int32),
            pltpu.VMEM((256,), jnp.int32),
            pltpu.VMEM((nch, _CH), jnp.int32),
            pltpu.VMEM((_CH, d_model), jnp.float32),
            pltpu.VMEM((kpad * 3,), jnp.float32),
            pltpu.SemaphoreType.DMA,
        ],
    )
    def sc_kernel(p_hbm, c_hbm, sel_hbm, outp_hbm, outc_hbm,
                  c_loc, sel_loc, keys_a, idx_a, keys_b, idx_b,
                  hist, offs, gidx, pbuf, cbuf, sem):
        wid = lax.axis_index("s") * _NC + lax.axis_index("c")
        iota = lax.iota(jnp.int32, _L)
        pltpu.sync_copy(sel_hbm, sel_loc)
        for rep in range(reps):
            b = wid * reps + rep
            pltpu.sync_copy(c_hbm.at[b], c_loc)
            selv = plsc.load_gather(sel_loc, [jnp.full((_L,), b, jnp.int32)])
            s3 = selv * 3
            sx = plsc.load_gather(c_loc, [s3])
            sy = plsc.load_gather(c_loc, [s3 + 1])
            sz = plsc.load_gather(c_loc, [s3 + 2])

            def d2_body(j, carry):
                ind = j * _L + iota
                i3 = ind * 3
                x = plsc.load_gather(c_loc, [i3])
                y = plsc.load_gather(c_loc, [i3 + 1])
                z = plsc.load_gather(c_loc, [i3 + 2])
                dx = sx - x
                dy = sy - y
                dz = sz - z
                d2 = dx * dx + dy * dy + dz * dz
                keys_a[pl.ds(j * _L, _L)] = plsc.bitcast(d2, jnp.int32)
                idx_a[pl.ds(j * _L, _L)] = ind
                return carry

            lax.fori_loop(0, n_vecs, d2_body, 0)

            for p in range(4):
                src_k, src_i = (keys_a, idx_a) if p % 2 == 0 else (keys_b, idx_b)
                dst_k, dst_i = (keys_b, idx_b) if p % 2 == 0 else (keys_a, idx_a)
                sh = 8 * p

                def zero_body(h, carry):
                    hist[pl.ds(h * _L, _L)] = jnp.zeros((_L,), jnp.int32)
                    return carry

                lax.fori_loop(0, 256 // _L, zero_body, 0)

                def hist_body(j, carry, src_k=src_k, sh=sh):
                    k = src_k[pl.ds(j * _L, _L)]
                    dig = (k >> sh) & 0xFF
                    cnt, last = plsc.scan_count(dig)
                    plsc.addupdate_scatter(hist, [dig], cnt, mask=last)
                    return carry

                lax.fori_loop(0, n_vecs, hist_body, 0)

                def scan_body(h, carry):
                    hv = hist[pl.ds(h * _L, _L)]
                    cs = plsc.cumsum(hv)
                    offs[pl.ds(h * _L, _L)] = cs - hv + carry
                    return carry + jnp.sum(hv)

                lax.fori_loop(0, 256 // _L, scan_body, jnp.int32(0))

                def perm_body(j, carry, src_k=src_k, src_i=src_i,
                              dst_k=dst_k, dst_i=dst_i, sh=sh):
                    k = src_k[pl.ds(j * _L, _L)]
                    v = src_i[pl.ds(j * _L, _L)]
                    dig = (k >> sh) & 0xFF
                    cnt, last = plsc.scan_count(dig)
                    pos = plsc.load_gather(offs, [dig]) + cnt - 1
                    plsc.store_scatter(dst_k, [pos], k)
                    plsc.store_scatter(dst_i, [pos], v)
                    plsc.addupdate_scatter(offs, [dig], cnt, mask=last)
                    return carry

                lax.fori_loop(0, n_vecs, perm_body, 0)

            base_row = b * n

            def gidx_body(j, carry):
                i16 = idx_a[pl.ds(j * _L, _L)]
                r = j // (_CH // _L)
                cc = (j % (_CH // _L)) * _L
                gidx[r, pl.ds(cc, _L)] = i16 + base_row
                g3 = i16 * 3
                x = plsc.load_gather(c_loc, [g3])
                y = plsc.load_gather(c_loc, [g3 + 1])
                z = plsc.load_gather(c_loc, [g3 + 2])
                o3 = (j * _L + iota) * 3
                plsc.store_scatter(cbuf, [o3], x)
                plsc.store_scatter(cbuf, [o3 + 1], y)
                plsc.store_scatter(cbuf, [o3 + 2], z)
                return carry

            lax.fori_loop(0, kpad // _L, gidx_body, 0)
            pltpu.sync_copy(cbuf, outc_hbm.at[b])
            for ch in range(nch):
                pltpu.async_copy(p_hbm.at[gidx.at[ch]], pbuf, sem).wait()
                pltpu.sync_copy(
                    pbuf, outp_hbm.at[pl.ds(b * kpad + ch * _CH, _CH)])

    return sc_kernel


def kernel(patches, centers):
    B, C, n, G, _ = patches.shape
    bs = B * C
    d_model = G * 3
    lo = int(0.25 * n)
    hi = int(0.75 * n)
    K = int(np.random.default_rng(0).integers(lo, hi))
    kpad = -(-K // _CH) * _CH

    p_flat = patches.reshape(bs * n, d_model)
    c_flat = centers.reshape(bs, n * 3)
    rk = jax.random.key(42)
    sel = jax.random.randint(rk, (bs, 1), 0, n).reshape(bs).astype(jnp.int32)

    outp, outc = _make_sc_kernel(bs, n, d_model, kpad)(p_flat, c_flat, sel)
    cropped_patches = outp.reshape(bs, kpad, G, 3)[:, :K].reshape(B, C, K, G, 3)
    cropped_centers = outc.reshape(bs, kpad, 3)[:, :K].reshape(B, C, K, 3)
    return cropped_patches, cropped_centers


# speedup vs baseline: 22.1177x; 2.4191x over previous
import functools

import numpy as np
import jax
import jax.numpy as jnp
from jax import lax
from jax.experimental import pallas as pl
from jax.experimental.pallas import tpu as pltpu
from jax.experimental.pallas import tpu_sc as plsc

_NC, _NS, _L = 2, 16, 16


@functools.lru_cache(maxsize=None)
def _make_sc_kernel(bs, n, K):
    kpad = -(-K // 128) * 128
    n_vecs = n // _L
    k_vecs = kpad // _L
    reps = bs // (_NC * _NS)
    n_slabs = 96 // 8
    mesh = plsc.VectorSubcoreMesh(
        core_axis_name="c", subcore_axis_name="s",
        num_cores=_NC, num_subcores=_NS)

    @functools.partial(
        pl.kernel,
        out_type=(
            jax.ShapeDtypeStruct((bs * 96, kpad), jnp.float32),
            jax.ShapeDtypeStruct((bs * 3, kpad), jnp.float32),
        ),
        mesh=mesh,
        compiler_params=pltpu.CompilerParams(
            needs_layout_passes=False, use_tc_tiling_on_sc=False),
        scratch_types=[
            pltpu.VMEM((3 * n,), jnp.float32),
            pltpu.VMEM((bs,), jnp.int32),
            pltpu.VMEM((n,), jnp.int32),
            pltpu.VMEM((n,), jnp.int32),
            pltpu.VMEM((n,), jnp.int32),
            pltpu.VMEM((n,), jnp.int32),
            pltpu.VMEM((256,), jnp.int32),
            pltpu.VMEM((256,), jnp.int32),
            pltpu.VMEM((8, n), jnp.float32),
            pltpu.VMEM((8, kpad), jnp.float32),
            pltpu.VMEM((3 * kpad,), jnp.float32),
            pltpu.SemaphoreType.DMA,
        ],
    )
    def sc_kernel(p_hbm, c_hbm, sel_hbm, outp_hbm, outc_hbm,
                  cxyz, sel_loc, keys_a, idx_a, keys_b, idx_b,
                  hist, offs, slab, obuf, cbuf, sem):
        wid = lax.axis_index("s") * _NC + lax.axis_index("c")
        iota = lax.iota(jnp.int32, _L)
        pltpu.sync_copy(sel_hbm, sel_loc)
        for rep in range(reps):
            b2 = wid * reps + rep
            b = b2 // 2
            c = b2 % 2
            for d in range(3):
                pltpu.sync_copy(c_hbm.at[(b * 3 + d) * 2 + c],
                                cxyz.at[pl.ds(d * n, n)])
            selv = plsc.load_gather(sel_loc, [jnp.full((_L,), b2, jnp.int32)])
            sx = plsc.load_gather(cxyz, [selv])
            sy = plsc.load_gather(cxyz, [selv + n])
            sz = plsc.load_gather(cxyz, [selv + 2 * n])

            @plsc.parallel_loop(0, n_vecs, unroll=4)
            def d2_body(j):
                o = j * _L
                dx = sx - cxyz[pl.ds(o, _L)]
                dy = sy - cxyz[pl.ds(n + o, _L)]
                dz = sz - cxyz[pl.ds(2 * n + o, _L)]
                d2 = dx * dx + dy * dy + dz * dz
                keys_a[pl.ds(o, _L)] = plsc.bitcast(d2, jnp.int32)
                idx_a[pl.ds(o, _L)] = o + iota

            for p in range(4):
                src_k, src_i = (keys_a, idx_a) if p % 2 == 0 else (keys_b, idx_b)
                dst_k, dst_i = (keys_b, idx_b) if p % 2 == 0 else (keys_a, idx_a)
                sh = 8 * p

                @plsc.parallel_loop(0, 256 // _L, unroll=4)
                def zero_body(h):
                    hist[pl.ds(h * _L, _L)] = jnp.zeros((_L,), jnp.int32)

                def hist_body(j, carry, src_k=src_k, sh=sh):
                    k = src_k[pl.ds(j * _L, _L)]
                    dig = (k >> sh) & 0xFF
                    cnt, last = plsc.scan_count(dig)
                    plsc.addupdate_scatter(hist, [dig], cnt, mask=last)
                    return carry

                lax.fori_loop(0, n_vecs, hist_body, 0)

                def scan_body(h, carry):
                    hv = hist[pl.ds(h * _L, _L)]
                    cs = plsc.cumsum(hv)
                    offs[pl.ds(h * _L, _L)] = cs - hv + carry
                    return carry + jnp.sum(hv)

                lax.fori_loop(0, 256 // _L, scan_body, jnp.int32(0))

                def perm_body(j, carry, src_k=src_k, src_i=src_i,
                              dst_k=dst_k, dst_i=dst_i, sh=sh):
                    k = src_k[pl.ds(j * _L, _L)]
                    v = src_i[pl.ds(j * _L, _L)]
                    dig = (k >> sh) & 0xFF
                    cnt, last = plsc.scan_count(dig)
                    pos = plsc.load_gather(offs, [dig]) + cnt - 1
                    plsc.store_scatter(dst_k, [pos], k)
                    plsc.store_scatter(dst_i, [pos], v)
                    plsc.addupdate_scatter(offs, [dig], cnt, mask=last)
                    return carry

                lax.fori_loop(0, n_vecs, perm_body, 0)

            @plsc.parallel_loop(0, k_vecs, unroll=4)
            def cgather_body(j):
                idxv = idx_a[pl.ds(j * _L, _L)]
                for d in range(3):
                    v = plsc.load_gather(cxyz, [idxv + d * n])
                    cbuf[pl.ds(d * kpad + j * _L, _L)] = v

            for d in range(3):
                pltpu.sync_copy(cbuf.at[pl.ds(d * kpad, kpad)],
                                outc_hbm.at[(b * 3 + d) * 2 + c])

            row0 = b2 * 96
            for s in range(n_slabs):
                pltpu.sync_copy(p_hbm.at[pl.ds(row0 + s * 8, 8)], slab)

                @plsc.parallel_loop(0, k_vecs, unroll=2)
                def pgather_body(j):
                    idxv = idx_a[pl.ds(j * _L, _L)]
                    for g in range(8):
                        v = plsc.load_gather(
                            slab, [jnp.full((_L,), g, jnp.int32), idxv])
                        obuf[g, pl.ds(j * _L, _L)] = v

                pltpu.sync_copy(
                    obuf, outp_hbm.at[pl.ds(row0 + s * 8, 8)])

    return sc_kernel


def kernel(patches, centers):
    B, C, n, G, _ = patches.shape
    bs = B * C
    lo = int(0.25 * n)
    hi = int(0.75 * n)
    K = int(np.random.default_rng(0).integers(lo, hi))

    p_t = patches.transpose(0, 1, 4, 3, 2).reshape(bs * 3 * G, n)
    c_t = centers.transpose(0, 3, 1, 2).reshape(bs * 3, n)
    rk = jax.random.key(42)
    sel = jax.random.randint(rk, (bs, 1), 0, n).reshape(bs).astype(jnp.int32)

    kpad = -(-K // 128) * 128
    outp, outc = _make_sc_kernel(bs, n, K)(p_t, c_t, sel)
    cropped_patches = (
        outp.reshape(B, C, 3, G, kpad)[..., :K].transpose(0, 1, 4, 3, 2))
    cropped_centers = (
        outc.reshape(B, 3, C, kpad)[..., :K].transpose(0, 2, 3, 1))
    return cropped_patches, cropped_centers


# speedup vs baseline: 32.2948x; 1.4601x over previous
import functools

import numpy as np
import jax
import jax.numpy as jnp
from jax import lax
from jax.experimental import pallas as pl
from jax.experimental.pallas import tpu as pltpu
from jax.experimental.pallas import tpu_sc as plsc

_NC, _NS, _L = 2, 16, 16


@functools.lru_cache(maxsize=None)
def _make_sc_kernel(bs, n, K):
    kpad = -(-K // 128) * 128
    n_vecs = n // _L
    k_vecs = kpad // _L
    nt = n // 128
    kt = kpad // 128
    reps = bs // (_NC * _NS)
    mesh = plsc.VectorSubcoreMesh(
        core_axis_name="c", subcore_axis_name="s",
        num_cores=_NC, num_subcores=_NS)

    @functools.partial(
        pl.kernel,
        out_type=(
            jax.ShapeDtypeStruct((bs * 12, kt, 8, 128), jnp.float32),
            jax.ShapeDtypeStruct((bs * 3, kpad), jnp.float32),
        ),
        mesh=mesh,
        compiler_params=pltpu.CompilerParams(
            needs_layout_passes=False, use_tc_tiling_on_sc=False),
        scratch_types=[
            pltpu.VMEM((3, nt, 2, 128), jnp.float32),
            pltpu.VMEM((bs,), jnp.int32),
            pltpu.VMEM((n,), jnp.int32),
            pltpu.VMEM((n,), jnp.int32),
            pltpu.VMEM((n,), jnp.int32),
            pltpu.VMEM((n,), jnp.int32),
            pltpu.VMEM((256,), jnp.int32),
            pltpu.VMEM((256,), jnp.int32),
            pltpu.VMEM((kpad,), jnp.int32),
            pltpu.VMEM((kpad,), jnp.int32),
            pltpu.VMEM((nt, 8, 128), jnp.float32),
            pltpu.VMEM((kt, 8, 128), jnp.float32),
            pltpu.VMEM((3 * kpad,), jnp.float32),
            pltpu.SemaphoreType.DMA,
        ],
    )
    def sc_kernel(p_hbm, c_hbm, sel_hbm, outp_hbm, outc_hbm,
                  cb, sel_loc, keys_a, idx_a, keys_b, idx_b,
                  hist, offs, ihi, ilo, slab, obuf, cbuf, sem):
        wid = lax.axis_index("s") * _NC + lax.axis_index("c")
        iota = lax.iota(jnp.int32, _L)
        zeros = jnp.zeros((_L,), jnp.int32)
        pltpu.sync_copy(sel_hbm, sel_loc)
        for rep in range(reps):
            b2 = wid * reps + rep
            b = b2 // 2
            c = b2 % 2
            cv = jnp.full((_L,), c, jnp.int32)
            for d in range(3):
                pltpu.sync_copy(c_hbm.at[b * 3 + d], cb.at[d])
            selv = plsc.load_gather(sel_loc, [jnp.full((_L,), b2, jnp.int32)])
            shi = selv >> 7
            slo = selv & 127
            sx = plsc.load_gather(cb, [zeros, shi, cv, slo])
            sy = plsc.load_gather(cb, [zeros + 1, shi, cv, slo])
            sz = plsc.load_gather(cb, [zeros + 2, shi, cv, slo])

            @plsc.parallel_loop(0, n_vecs, unroll=4)
            def d2_body(j):
                t = j // 8
                k0 = (j % 8) * _L
                dx = sx - cb[0, t, c, pl.ds(k0, _L)]
                dy = sy - cb[1, t, c, pl.ds(k0, _L)]
                dz = sz - cb[2, t, c, pl.ds(k0, _L)]
                d2 = dx * dx + dy * dy + dz * dz
                keys_a[pl.ds(j * _L, _L)] = plsc.bitcast(d2, jnp.int32)
                idx_a[pl.ds(j * _L, _L)] = j * _L + iota

            for p in range(4):
                src_k, src_i = (keys_a, idx_a) if p % 2 == 0 else (keys_b, idx_b)
                dst_k, dst_i = (keys_b, idx_b) if p % 2 == 0 else (keys_a, idx_a)
                sh = 8 * p

                @plsc.parallel_loop(0, 256 // _L, unroll=4)
                def zero_body(h):
                    hist[pl.ds(h * _L, _L)] = jnp.zeros((_L,), jnp.int32)

                def hist_body(j, carry, src_k=src_k, sh=sh):
                    k = src_k[pl.ds(j * _L, _L)]
                    dig = (k >> sh) & 0xFF
                    cnt, last = plsc.scan_count(dig)
                    plsc.addupdate_scatter(hist, [dig], cnt, mask=last)
                    return carry

                lax.fori_loop(0, n_vecs, hist_body, 0)

                def scan_body(h, carry):
                    hv = hist[pl.ds(h * _L, _L)]
                    cs = plsc.cumsum(hv)
                    offs[pl.ds(h * _L, _L)] = cs - hv + carry
                    return carry + jnp.sum(hv)

                lax.fori_loop(0, 256 // _L, scan_body, jnp.int32(0))

                def perm_body(j, carry, src_k=src_k, src_i=src_i,
                              dst_k=dst_k, dst_i=dst_i, sh=sh):
                    k = src_k[pl.ds(j * _L, _L)]
                    v = src_i[pl.ds(j * _L, _L)]
                    dig = (k >> sh) & 0xFF
                    cnt, last = plsc.scan_count(dig)
                    pos = plsc.load_gather(offs, [dig]) + cnt - 1
                    plsc.store_scatter(dst_k, [pos], k)
                    plsc.store_scatter(dst_i, [pos], v)
                    plsc.addupdate_scatter(offs, [dig], cnt, mask=last)
                    return carry

                lax.fori_loop(0, n_vecs, perm_body, 0)

            @plsc.parallel_loop(0, k_vecs, unroll=4)
            def cgather_body(j):
                idxv = idx_a[pl.ds(j * _L, _L)]
                hi = idxv >> 7
                lo = idxv & 127
                ihi[pl.ds(j * _L, _L)] = hi
                ilo[pl.ds(j * _L, _L)] = lo
                for d in range(3):
                    v = plsc.load_gather(cb, [zeros + d, hi, cv, lo])
                    cbuf[pl.ds(d * kpad + j * _L, _L)] = v

            for d in range(3):
                pltpu.sync_copy(cbuf.at[pl.ds(d * kpad, kpad)],
                                outc_hbm.at[(b * 3 + d) * 2 + c])

            for s in range(12):
                rg = b2 * 12 + s
                pltpu.sync_copy(p_hbm.at[rg], slab)

                @plsc.parallel_loop(0, k_vecs, unroll=2)
                def pgather_body(j):
                    hi = ihi[pl.ds(j * _L, _L)]
                    lo = ilo[pl.ds(j * _L, _L)]
                    t = j // 8
                    k0 = (j % 8) * _L
                    for g in range(8):
                        v = plsc.load_gather(
                            slab, [hi, jnp.full((_L,), g, jnp.int32), lo])
                        obuf[t, g, pl.ds(k0, _L)] = v

                pltpu.sync_copy(obuf, outp_hbm.at[rg])

    return sc_kernel


def kernel(patches, centers):
    B, C, n, G, _ = patches.shape
    bs = B * C
    lo = int(0.25 * n)
    hi = int(0.75 * n)
    K = int(np.random.default_rng(0).integers(lo, hi))
    kpad = -(-K // 128) * 128

    p4 = (patches.transpose(0, 1, 4, 3, 2)
          .reshape(bs * 12, 8, n // 128, 128).transpose(0, 2, 1, 3))
    c4 = (centers.transpose(0, 3, 1, 2)
          .reshape(bs * 3 // 2, 2, n // 128, 128).transpose(0, 2, 1, 3))
    rk = jax.random.key(42)
    sel = jax.random.randint(rk, (bs, 1), 0, n).reshape(bs).astype(jnp.int32)

    outp4, outc = _make_sc_kernel(bs, n, K)(p4, c4, sel)
    cropped_patches = (
        outp4.transpose(0, 2, 1, 3).reshape(B, C, 3, G, kpad)
        [..., :K].transpose(0, 1, 4, 3, 2))
    cropped_centers = (
        outc.reshape(B, 3, C, kpad)[..., :K].transpose(0, 2, 3, 1))
    return cropped_patches, cropped_centers


# speedup vs baseline: 39.0245x; 1.2084x over previous
import functools

import numpy as np
import jax
import jax.numpy as jnp
from jax import lax
from jax.experimental import pallas as pl
from jax.experimental.pallas import tpu as pltpu
from jax.experimental.pallas import tpu_sc as plsc

_NC, _NS, _L = 2, 16, 16


@functools.lru_cache(maxsize=None)
def _make_sc_kernel(bs, n, K):
    kpad = -(-K // 128) * 128
    n_vecs = n // _L
    k_vecs = kpad // _L
    nt = n // 128
    kt = kpad // 128
    reps = bs // (_NC * _NS)
    mesh = plsc.VectorSubcoreMesh(
        core_axis_name="c", subcore_axis_name="s",
        num_cores=_NC, num_subcores=_NS)

    @functools.partial(
        pl.kernel,
        out_type=(
            jax.ShapeDtypeStruct((bs * 12, kt, 8, 128), jnp.float32),
            jax.ShapeDtypeStruct((bs * 3, kpad), jnp.float32),
        ),
        mesh=mesh,
        compiler_params=pltpu.CompilerParams(
            needs_layout_passes=False, use_tc_tiling_on_sc=False),
        scratch_types=[
            pltpu.VMEM((3, nt, 2, 128), jnp.float32),
            pltpu.VMEM((bs,), jnp.int32),
            pltpu.VMEM((n,), jnp.int32),
            pltpu.VMEM((n,), jnp.int32),
            pltpu.VMEM((n,), jnp.int32),
            pltpu.VMEM((n,), jnp.int32),
            pltpu.VMEM((256,), jnp.int32),
            pltpu.VMEM((256,), jnp.int32),
            pltpu.VMEM((kpad,), jnp.int32),
            pltpu.VMEM((kpad,), jnp.int32),
            pltpu.VMEM((nt, 8, 128), jnp.float32),
            pltpu.VMEM((nt, 8, 128), jnp.float32),
            pltpu.VMEM((kt, 8, 128), jnp.float32),
            pltpu.VMEM((kt, 8, 128), jnp.float32),
            pltpu.VMEM((3 * kpad,), jnp.float32),
            pltpu.SemaphoreType.DMA,
            pltpu.SemaphoreType.DMA,
            pltpu.SemaphoreType.DMA,
            pltpu.SemaphoreType.DMA,
        ],
    )
    def sc_kernel(p_hbm, c_hbm, sel_hbm, outp_hbm, outc_hbm,
                  cb, sel_loc, keys_a, idx_a, keys_b, idx_b,
                  hist, offs, ihi, ilo, slab0, slab1, obuf0, obuf1,
                  cbuf, sin0, sin1, sout0, sout1):
        wid = lax.axis_index("s") * _NC + lax.axis_index("c")
        iota = lax.iota(jnp.int32, _L)
        zeros = jnp.zeros((_L,), jnp.int32)
        pltpu.sync_copy(sel_hbm, sel_loc)
        for rep in range(reps):
            b2 = wid * reps + rep
            b = b2 // 2
            c = b2 % 2
            cv = jnp.full((_L,), c, jnp.int32)
            for d in range(3):
                pltpu.sync_copy(c_hbm.at[b * 3 + d], cb.at[d])
            selv = plsc.load_gather(sel_loc, [jnp.full((_L,), b2, jnp.int32)])
            shi = selv >> 7
            slo = selv & 127
            sx = plsc.load_gather(cb, [zeros, shi, cv, slo])
            sy = plsc.load_gather(cb, [zeros + 1, shi, cv, slo])
            sz = plsc.load_gather(cb, [zeros + 2, shi, cv, slo])

            @plsc.parallel_loop(0, n_vecs, unroll=4)
            def d2_body(j):
                t = j // 8
                k0 = (j % 8) * _L
                dx = sx - cb[0, t, c, pl.ds(k0, _L)]
                dy = sy - cb[1, t, c, pl.ds(k0, _L)]
                dz = sz - cb[2, t, c, pl.ds(k0, _L)]
                d2 = dx * dx + dy * dy + dz * dz
                keys_a[pl.ds(j * _L, _L)] = plsc.bitcast(d2, jnp.int32)
                idx_a[pl.ds(j * _L, _L)] = j * _L + iota

            for p in range(4):
                src_k, src_i = (keys_a, idx_a) if p % 2 == 0 else (keys_b, idx_b)
                dst_k, dst_i = (keys_b, idx_b) if p % 2 == 0 else (keys_a, idx_a)
                sh = 8 * p

                @plsc.parallel_loop(0, 256 // _L, unroll=4)
                def zero_body(h):
                    hist[pl.ds(h * _L, _L)] = jnp.zeros((_L,), jnp.int32)

                def hist_body(j, carry, src_k=src_k, sh=sh):
                    k = src_k[pl.ds(j * _L, _L)]
                    dig = (k >> sh) & 0xFF
                    cnt, last = plsc.scan_count(dig)
                    plsc.addupdate_scatter(hist, [dig], cnt, mask=last)
                    return carry

                lax.fori_loop(0, n_vecs, hist_body, 0, unroll=4)

                def scan_body(h, carry):
                    hv = hist[pl.ds(h * _L, _L)]
                    cs = plsc.cumsum(hv)
                    offs[pl.ds(h * _L, _L)] = cs - hv + carry
                    return carry + jnp.sum(hv)

                lax.fori_loop(0, 256 // _L, scan_body, jnp.int32(0), unroll=2)

                def perm_body(j, carry, src_k=src_k, src_i=src_i,
                              dst_k=dst_k, dst_i=dst_i, sh=sh):
                    k = src_k[pl.ds(j * _L, _L)]
                    v = src_i[pl.ds(j * _L, _L)]
                    dig = (k >> sh) & 0xFF
                    cnt, last = plsc.scan_count(dig)
                    pos = plsc.load_gather(offs, [dig]) + cnt - 1
                    plsc.store_scatter(dst_k, [pos], k)
                    plsc.store_scatter(dst_i, [pos], v)
                    plsc.addupdate_scatter(offs, [dig], cnt, mask=last)
                    return carry

                lax.fori_loop(0, n_vecs, perm_body, 0, unroll=4)

            @plsc.parallel_loop(0, k_vecs, unroll=4)
            def cgather_body(j):
                idxv = idx_a[pl.ds(j * _L, _L)]
                hi = idxv >> 7
                lo = idxv & 127
                ihi[pl.ds(j * _L, _L)] = hi
                ilo[pl.ds(j * _L, _L)] = lo
                for d in range(3):
                    v = plsc.load_gather(cb, [zeros + d, hi, cv, lo])
                    cbuf[pl.ds(d * kpad + j * _L, _L)] = v

            for d in range(3):
                pltpu.sync_copy(cbuf.at[pl.ds(d * kpad, kpad)],
                                outc_hbm.at[(b * 3 + d) * 2 + c])

            slabs = (slab0, slab1)
            obufs = (obuf0, obuf1)
            sins = (sin0, sin1)
            souts = (sout0, sout1)
            in_copies = [None, None]
            out_copies = [None, None]
            in_copies[0] = pltpu.async_copy(
                p_hbm.at[b2 * 12], slabs[0], sins[0])
            for s in range(12):
                cur = s % 2
                if s + 1 < 12:
                    in_copies[1 - cur] = pltpu.async_copy(
                        p_hbm.at[b2 * 12 + s + 1], slabs[1 - cur],
                        sins[1 - cur])
                in_copies[cur].wait()
                if out_copies[cur] is not None:
                    out_copies[cur].wait()
                slab = slabs[cur]
                obuf = obufs[cur]

                @plsc.parallel_loop(0, k_vecs, unroll=2)
                def pgather_body(j, slab=slab, obuf=obuf):
                    hi = ihi[pl.ds(j * _L, _L)]
                    lo = ilo[pl.ds(j * _L, _L)]
                    t = j // 8
                    k0 = (j % 8) * _L
                    for g in range(8):
                        v = plsc.load_gather(
                            slab, [hi, jnp.full((_L,), g, jnp.int32), lo])
                        obuf[t, g, pl.ds(k0, _L)] = v

                out_copies[cur] = pltpu.async_copy(
                    obufs[cur], outp_hbm.at[b2 * 12 + s], souts[cur])
            out_copies[0].wait()
            out_copies[1].wait()

    return sc_kernel


def kernel(patches, centers):
    B, C, n, G, _ = patches.shape
    bs = B * C
    lo = int(0.25 * n)
    hi = int(0.75 * n)
    K = int(np.random.default_rng(0).integers(lo, hi))
    kpad = -(-K // 128) * 128

    p4 = (patches.transpose(0, 1, 4, 3, 2)
          .reshape(bs * 12, 8, n // 128, 128).transpose(0, 2, 1, 3))
    c4 = (centers.transpose(0, 3, 1, 2)
          .reshape(bs * 3 // 2, 2, n // 128, 128).transpose(0, 2, 1, 3))
    rk = jax.random.key(42)
    sel = jax.random.randint(rk, (bs, 1), 0, n).reshape(bs).astype(jnp.int32)

    outp4, outc = _make_sc_kernel(bs, n, K)(p4, c4, sel)
    cropped_patches = (
        outp4.transpose(0, 2, 1, 3).reshape(B, C, 3, G, kpad)
        [..., :K].transpose(0, 1, 4, 3, 2))
    cropped_centers = (
        outc.reshape(B, 3, C, kpad)[..., :K].transpose(0, 2, 3, 1))
    return cropped_patches, cropped_centers


# speedup vs baseline: 43.0999x; 1.1044x over previous
import functools

import numpy as np
import jax
import jax.numpy as jnp
from jax import lax
from jax.experimental import pallas as pl
from jax.experimental.pallas import tpu as pltpu
from jax.experimental.pallas import tpu_sc as plsc

_NC, _NS, _L = 2, 16, 16


@functools.lru_cache(maxsize=None)
def _make_sc_kernel(bs, n, K):
    kpad = -(-K // 128) * 128
    n_vecs = n // _L
    k_vecs = kpad // _L
    nt = n // 128
    kt = kpad // 128
    reps = bs // (_NC * _NS)
    mesh = plsc.VectorSubcoreMesh(
        core_axis_name="c", subcore_axis_name="s",
        num_cores=_NC, num_subcores=_NS)

    @functools.partial(
        pl.kernel,
        out_type=(
            jax.ShapeDtypeStruct((bs * 12, kt, 8, 128), jnp.float32),
            jax.ShapeDtypeStruct((bs * 3, kpad), jnp.float32),
        ),
        mesh=mesh,
        compiler_params=pltpu.CompilerParams(
            needs_layout_passes=False, use_tc_tiling_on_sc=False),
        scratch_types=[
            pltpu.VMEM((3, nt, 2, 128), jnp.float32),
            pltpu.VMEM((bs,), jnp.int32),
            pltpu.VMEM((n,), jnp.int32),
            pltpu.VMEM((n,), jnp.int32),
            pltpu.VMEM((n,), jnp.int32),
            pltpu.VMEM((n,), jnp.int32),
            pltpu.VMEM((256,), jnp.int32),
            pltpu.VMEM((256,), jnp.int32),
            pltpu.VMEM((kpad,), jnp.int32),
            pltpu.VMEM((kpad,), jnp.int32),
            pltpu.VMEM((nt, 8, 128), jnp.float32),
            pltpu.VMEM((nt, 8, 128), jnp.float32),
            pltpu.VMEM((nt, 8, 128), jnp.float32),
            pltpu.VMEM((kt, 8, 128), jnp.float32),
            pltpu.VMEM((kt, 8, 128), jnp.float32),
            pltpu.VMEM((kt, 8, 128), jnp.float32),
            pltpu.VMEM((3 * kpad,), jnp.float32),
            pltpu.SemaphoreType.DMA,
            pltpu.SemaphoreType.DMA,
            pltpu.SemaphoreType.DMA,
            pltpu.SemaphoreType.DMA,
            pltpu.SemaphoreType.DMA,
            pltpu.SemaphoreType.DMA,
        ],
    )
    def sc_kernel(p_hbm, c_hbm, sel_hbm, outp_hbm, outc_hbm,
                  cb, sel_loc, keys_a, idx_a, keys_b, idx_b,
                  hist, offs, ihi, ilo, slab0, slab1, slab2,
                  obuf0, obuf1, obuf2, cbuf,
                  sin0, sin1, sin2, sout0, sout1, sout2):
        wid = lax.axis_index("s") * _NC + lax.axis_index("c")
        iota = lax.iota(jnp.int32, _L)
        zeros = jnp.zeros((_L,), jnp.int32)
        pltpu.sync_copy(sel_hbm, sel_loc)
        for rep in range(reps):
            b2 = wid * reps + rep
            b = b2 // 2
            c = b2 % 2
            cv = jnp.full((_L,), c, jnp.int32)
            for d in range(3):
                pltpu.sync_copy(c_hbm.at[b * 3 + d], cb.at[d])
            selv = plsc.load_gather(sel_loc, [jnp.full((_L,), b2, jnp.int32)])
            shi = selv >> 7
            slo = selv & 127
            sx = plsc.load_gather(cb, [zeros, shi, cv, slo])
            sy = plsc.load_gather(cb, [zeros + 1, shi, cv, slo])
            sz = plsc.load_gather(cb, [zeros + 2, shi, cv, slo])

            @plsc.parallel_loop(0, 256 // _L, unroll=4)
            def zero0_body(h):
                hist[pl.ds(h * _L, _L)] = jnp.zeros((_L,), jnp.int32)

            def d2_body(j, carry):
                t = j // 8
                k0 = (j % 8) * _L
                dx = sx - cb[0, t, c, pl.ds(k0, _L)]
                dy = sy - cb[1, t, c, pl.ds(k0, _L)]
                dz = sz - cb[2, t, c, pl.ds(k0, _L)]
                d2 = dx * dx + dy * dy + dz * dz
                k = plsc.bitcast(d2, jnp.int32)
                keys_a[pl.ds(j * _L, _L)] = k
                idx_a[pl.ds(j * _L, _L)] = j * _L + iota
                dig = k & 0xFF
                cnt, last = plsc.scan_count(dig)
                plsc.addupdate_scatter(hist, [dig], cnt, mask=last)
                return carry

            lax.fori_loop(0, n_vecs, d2_body, 0, unroll=4)

            for p in range(4):
                src_k, src_i = (keys_a, idx_a) if p % 2 == 0 else (keys_b, idx_b)
                dst_k, dst_i = (keys_b, idx_b) if p % 2 == 0 else (keys_a, idx_a)
                sh = 8 * p

                def scan_body(h, carry):
                    hv = hist[pl.ds(h * _L, _L)]
                    cs = plsc.cumsum(hv)
                    offs[pl.ds(h * _L, _L)] = cs - hv + carry
                    return carry + jnp.sum(hv)

                lax.fori_loop(0, 256 // _L, scan_body, jnp.int32(0), unroll=2)

                if p < 3:
                    @plsc.parallel_loop(0, 256 // _L, unroll=4)
                    def zero_body(h):
                        hist[pl.ds(h * _L, _L)] = jnp.zeros((_L,), jnp.int32)

                def perm_body(j, carry, src_k=src_k, src_i=src_i,
                              dst_k=dst_k, dst_i=dst_i, sh=sh, p=p):
                    k = src_k[pl.ds(j * _L, _L)]
                    v = src_i[pl.ds(j * _L, _L)]
                    dig = (k >> sh) & 0xFF
                    cnt, last = plsc.scan_count(dig)
                    pos = plsc.load_gather(offs, [dig]) + cnt - 1
                    plsc.store_scatter(dst_k, [pos], k)
                    plsc.store_scatter(dst_i, [pos], v)
                    plsc.addupdate_scatter(offs, [dig], cnt, mask=last)
                    if p < 3:
                        dig2 = (k >> (sh + 8)) & 0xFF
                        cnt2, last2 = plsc.scan_count(dig2)
                        plsc.addupdate_scatter(hist, [dig2], cnt2, mask=last2)
                    return carry

                lax.fori_loop(0, n_vecs, perm_body, 0, unroll=4)

            @plsc.parallel_loop(0, k_vecs, unroll=4)
            def cgather_body(j):
                idxv = idx_a[pl.ds(j * _L, _L)]
                hi = idxv >> 7
                lo = idxv & 127
                ihi[pl.ds(j * _L, _L)] = hi
                ilo[pl.ds(j * _L, _L)] = lo
                for d in range(3):
                    v = plsc.load_gather(cb, [zeros + d, hi, cv, lo])
                    cbuf[pl.ds(d * kpad + j * _L, _L)] = v

            for d in range(3):
                pltpu.sync_copy(cbuf.at[pl.ds(d * kpad, kpad)],
                                outc_hbm.at[(b * 3 + d) * 2 + c])

            nbuf = 3
            slabs = (slab0, slab1, slab2)
            obufs = (obuf0, obuf1, obuf2)
            sins = (sin0, sin1, sin2)
            souts = (sout0, sout1, sout2)
            in_copies = [None] * nbuf
            out_copies = [None] * nbuf
            for w in range(nbuf - 1):
                in_copies[w] = pltpu.async_copy(
                    p_hbm.at[b2 * 12 + w], slabs[w], sins[w])
            for s in range(12):
                cur = s % nbuf
                nxt = (s + nbuf - 1) % nbuf
                if s + nbuf - 1 < 12:
                    if out_copies[nxt] is not None:
                        out_copies[nxt].wait()
                        out_copies[nxt] = None
                    in_copies[nxt] = pltpu.async_copy(
                        p_hbm.at[b2 * 12 + s + nbuf - 1], slabs[nxt],
                        sins[nxt])
                in_copies[cur].wait()
                if out_copies[cur] is not None:
                    out_copies[cur].wait()
                slab = slabs[cur]
                obuf = obufs[cur]

                @plsc.parallel_loop(0, k_vecs, unroll=2)
                def pgather_body(j, slab=slab, obuf=obuf):
                    hi = ihi[pl.ds(j * _L, _L)]
                    lo = ilo[pl.ds(j * _L, _L)]
                    t = j // 8
                    k0 = (j % 8) * _L
                    for g in range(8):
                        v = plsc.load_gather(
                            slab, [hi, jnp.full((_L,), g, jnp.int32), lo])
                        obuf[t, g, pl.ds(k0, _L)] = v

                out_copies[cur] = pltpu.async_copy(
                    obufs[cur], outp_hbm.at[b2 * 12 + s], souts[cur])
            for oc in out_copies:
                oc.wait()

    return sc_kernel


def kernel(patches, centers):
    B, C, n, G, _ = patches.shape
    bs = B * C
    lo = int(0.25 * n)
    hi = int(0.75 * n)
    K = int(np.random.default_rng(0).integers(lo, hi))
    kpad = -(-K // 128) * 128

    p4 = (patches.transpose(0, 1, 4, 3, 2)
          .reshape(bs * 12, 8, n // 128, 128).transpose(0, 2, 1, 3))
    c4 = (centers.transpose(0, 3, 1, 2)
          .reshape(bs * 3 // 2, 2, n // 128, 128).transpose(0, 2, 1, 3))
    rk = jax.random.key(42)
    sel = jax.random.randint(rk, (bs, 1), 0, n).reshape(bs).astype(jnp.int32)

    outp4, outc = _make_sc_kernel(bs, n, K)(p4, c4, sel)
    cropped_patches = (
        outp4.transpose(0, 2, 1, 3).reshape(B, C, 3, G, kpad)
        [..., :K].transpose(0, 1, 4, 3, 2))
    cropped_centers = (
        outc.reshape(B, 3, C, kpad)[..., :K].transpose(0, 2, 3, 1))
    return cropped_patches, cropped_centers
